# baseline (speedup 1.0000x reference)
import functools

import jax
import jax.numpy as jnp
from jax import lax
from jax.experimental import pallas as pl
from jax.experimental.pallas import tpu as pltpu
from jax.experimental.pallas import tpu_sc as plsc

B, S, C = 4, 4096, 1024
BS = 256
NW = 32
ROWS_PER_W = S // NW
CH = 64


def _add_kernel(img_ref, aud_ref, pi_ref, pa_ref, oi_ref, oa_ref):
    pi = pi_ref[...]
    pa = pa_ref[...]
    oi_ref[...] = img_ref[...] + jnp.broadcast_to(pi[None], (B, BS, C))
    oa_ref[...] = aud_ref[...] + jnp.broadcast_to(pa[None], (B, BS, C))


def _tc_add(image, audio, pos_image, pos_audio):
    grid = (S // BS,)
    in_spec3 = pl.BlockSpec((B, BS, C), lambda s: (0, s, 0))
    in_spec2 = pl.BlockSpec((BS, C), lambda s: (s, 0))
    out_spec = pl.BlockSpec((B, BS, C), lambda s: (0, s, 0))
    out_shape = jax.ShapeDtypeStruct((B, S, C), jnp.float32)
    return pl.pallas_call(
        _add_kernel,
        grid=grid,
        in_specs=[in_spec3, in_spec3, in_spec2, in_spec2],
        out_specs=[out_spec, out_spec],
        out_shape=[out_shape, out_shape],
    )(image, audio, pos_image, pos_audio)


@functools.partial(
    pl.kernel,
    mesh=plsc.VectorSubcoreMesh(core_axis_name="c", subcore_axis_name="s"),
    out_type=[jax.ShapeDtypeStruct((B, S, C), jnp.float32),
              jax.ShapeDtypeStruct((B, S, C), jnp.float32)],
    scratch_types=[pltpu.VMEM((CH, C), jnp.float32),
                   pltpu.SemaphoreType.DMA],
)
def _sc_broadcast(pi_hbm, pa_hbm, pei_hbm, pea_hbm, buf, sem):
    wid = lax.axis_index("s") * 2 + lax.axis_index("c")
    base = wid * ROWS_PER_W
    for pos_hbm, pe_hbm in ((pi_hbm, pei_hbm), (pa_hbm, pea_hbm)):
        for chunk in range(ROWS_PER_W // CH):
            r = base + chunk * CH
            pltpu.sync_copy(pos_hbm.at[pl.ds(r, CH), :], buf)
            for b in range(B):
                pltpu.async_copy(buf, pe_hbm.at[b, pl.ds(r, CH), :], sem)
            for _ in range(B):
                pltpu.make_async_copy(buf, pe_hbm.at[0, pl.ds(r, CH), :],
                                      sem).wait()


def kernel(image, audio, pos_image, pos_audio):
    out_image, out_audio = _tc_add(image, audio, pos_image, pos_audio)
    pe_image, pe_audio = _sc_broadcast(pos_image, pos_audio)
    return (out_image, out_audio, pe_image, pe_audio)


# speedup vs baseline: 1.0086x; 1.0086x over previous
import functools

import jax
import jax.numpy as jnp
from jax import lax
from jax.experimental import pallas as pl
from jax.experimental.pallas import tpu as pltpu
from jax.experimental.pallas import tpu_sc as plsc

B, S, C = 4, 4096, 1024
BS = 256
NW = 32
ROWS_PER_W = S // NW
CH = 32


def _add_kernel(img_ref, aud_ref, pi_ref, pa_ref, oi_ref, oa_ref):
    pi = pi_ref[...]
    pa = pa_ref[...]
    oi_ref[...] = img_ref[...] + jnp.broadcast_to(pi[None], (B, BS, C))
    oa_ref[...] = aud_ref[...] + jnp.broadcast_to(pa[None], (B, BS, C))


def _tc_add(image, audio, pos_image, pos_audio):
    grid = (S // BS,)
    in_spec3 = pl.BlockSpec((B, BS, C), lambda s: (0, s, 0))
    in_spec2 = pl.BlockSpec((BS, C), lambda s: (s, 0))
    out_spec = pl.BlockSpec((B, BS, C), lambda s: (0, s, 0))
    out_shape = jax.ShapeDtypeStruct((B, S, C), jnp.float32)
    return pl.pallas_call(
        _add_kernel,
        grid=grid,
        in_specs=[in_spec3, in_spec3, in_spec2, in_spec2],
        out_specs=[out_spec, out_spec],
        out_shape=[out_shape, out_shape],
    )(image, audio, pos_image, pos_audio)


@functools.partial(
    pl.kernel,
    mesh=plsc.VectorSubcoreMesh(core_axis_name="c", subcore_axis_name="s"),
    out_type=[jax.ShapeDtypeStruct((B, S, C), jnp.float32),
              jax.ShapeDtypeStruct((B, S, C), jnp.float32)],
    scratch_types=[pltpu.VMEM((CH, C), jnp.float32),
                   pltpu.VMEM((CH, C), jnp.float32),
                   pltpu.SemaphoreType.DMA,
                   pltpu.SemaphoreType.DMA],
)
def _sc_broadcast(pi_hbm, pa_hbm, pei_hbm, pea_hbm, buf0, buf1, rsem, wsem):
    wid = lax.axis_index("s") * 2 + lax.axis_index("c")
    base = wid * ROWS_PER_W
    bufs = (buf0, buf1)
    chunks = []
    for pos_hbm, pe_hbm in ((pi_hbm, pei_hbm), (pa_hbm, pea_hbm)):
        for chunk in range(ROWS_PER_W // CH):
            chunks.append((pos_hbm, pe_hbm, base + chunk * CH))
    n = len(chunks)
    pltpu.async_copy(chunks[0][0].at[pl.ds(chunks[0][2], CH), :], bufs[0],
                     rsem)
    for i in range(n):
        pos_hbm, pe_hbm, r = chunks[i]
        buf = bufs[i % 2]
        pltpu.make_async_copy(pos_hbm.at[pl.ds(r, CH), :], buf, rsem).wait()
        if i + 1 < n:
            npos, _, nr = chunks[i + 1]
            nbuf = bufs[(i + 1) % 2]
            if i >= 1:
                ppos, ppe, pr = chunks[i - 1]
                for _ in range(B):
                    pltpu.make_async_copy(nbuf, ppe.at[0, pl.ds(pr, CH), :],
                                          wsem).wait()
            pltpu.async_copy(npos.at[pl.ds(nr, CH), :], nbuf, rsem)
        for b in range(B):
            pltpu.async_copy(buf, pe_hbm.at[b, pl.ds(r, CH), :], wsem)
    for i in (n - 2, n - 1):
        pos_hbm, pe_hbm, r = chunks[i]
        for _ in range(B):
            pltpu.make_async_copy(bufs[i % 2], pe_hbm.at[0, pl.ds(r, CH), :],
                                  wsem).wait()


def kernel(image, audio, pos_image, pos_audio):
    out_image, out_audio = _tc_add(image, audio, pos_image, pos_audio)
    pe_image, pe_audio = _sc_broadcast(pos_image, pos_audio)
    return (out_image, out_audio, pe_image, pe_audio)


# speedup vs baseline: 1.0544x; 1.0454x over previous
import functools

import jax
import jax.numpy as jnp
from jax import lax
from jax.experimental import pallas as pl
from jax.experimental.pallas import tpu as pltpu
from jax.experimental.pallas import tpu_sc as plsc

B, S, C = 4, 4096, 1024
BS = 256
NW = 32
ROWS_PER_W = S // NW
CH = 32


def _add_kernel(img_ref, aud_ref, pi_ref, pa_ref, oi_ref, oa_ref, pea_ref):
    pi = pi_ref[...]
    pa = pa_ref[...]
    pe_a = jnp.broadcast_to(pa[None], (B, BS, C))
    oi_ref[...] = img_ref[...] + jnp.broadcast_to(pi[None], (B, BS, C))
    oa_ref[...] = aud_ref[...] + pe_a
    pea_ref[...] = pe_a


def _tc_part(image, audio, pos_image, pos_audio):
    grid = (S // BS,)
    in_spec3 = pl.BlockSpec((B, BS, C), lambda s: (0, s, 0))
    in_spec2 = pl.BlockSpec((BS, C), lambda s: (s, 0))
    out_spec = pl.BlockSpec((B, BS, C), lambda s: (0, s, 0))
    out_shape = jax.ShapeDtypeStruct((B, S, C), jnp.float32)
    return pl.pallas_call(
        _add_kernel,
        grid=grid,
        in_specs=[in_spec3, in_spec3, in_spec2, in_spec2],
        out_specs=[out_spec, out_spec, out_spec],
        out_shape=[out_shape, out_shape, out_shape],
    )(image, audio, pos_image, pos_audio)


@functools.partial(
    pl.kernel,
    mesh=plsc.VectorSubcoreMesh(core_axis_name="c", subcore_axis_name="s"),
    out_type=jax.ShapeDtypeStruct((B, S, C), jnp.float32),
    scratch_types=[pltpu.VMEM((CH, C), jnp.float32),
                   pltpu.VMEM((CH, C), jnp.float32),
                   pltpu.SemaphoreType.DMA,
                   pltpu.SemaphoreType.DMA],
)
def _sc_broadcast(pos_hbm, pe_hbm, buf0, buf1, rsem, wsem):
    wid = lax.axis_index("s") * 2 + lax.axis_index("c")
    base = wid * ROWS_PER_W
    bufs = (buf0, buf1)
    rows = [base + c * CH for c in range(ROWS_PER_W // CH)]
    n = len(rows)
    pltpu.async_copy(pos_hbm.at[pl.ds(rows[0], CH), :], bufs[0], rsem)
    for i in range(n):
        r = rows[i]
        buf = bufs[i % 2]
        pltpu.make_async_copy(pos_hbm.at[pl.ds(r, CH), :], buf, rsem).wait()
        if i + 1 < n:
            nbuf = bufs[(i + 1) % 2]
            if i >= 1:
                pr = rows[i - 1]
                for _ in range(B):
                    pltpu.make_async_copy(nbuf, pe_hbm.at[0, pl.ds(pr, CH), :],
                                          wsem).wait()
            pltpu.async_copy(pos_hbm.at[pl.ds(rows[i + 1], CH), :], nbuf, rsem)
        for b in range(B):
            pltpu.async_copy(buf, pe_hbm.at[b, pl.ds(r, CH), :], wsem)
    for i in (n - 2, n - 1):
        for _ in range(B):
            pltpu.make_async_copy(bufs[i % 2], pe_hbm.at[0, pl.ds(rows[i], CH), :],
                                  wsem).wait()


def kernel(image, audio, pos_image, pos_audio):
    out_image, out_audio, pe_audio = _tc_part(image, audio, pos_image,
                                              pos_audio)
    pe_image = _sc_broadcast(pos_image)
    return (out_image, out_audio, pe_image, pe_audio)


# speedup vs baseline: 1.2154x; 1.1527x over previous
import jax
import jax.numpy as jnp
from jax.experimental import pallas as pl

B, S, C = 4, 4096, 1024
BS = 512


def _pe_kernel(img_ref, aud_ref, pi_ref, pa_ref,
               oi_ref, oa_ref, pei_ref, pea_ref):
    pi = pi_ref[...]
    pa = pa_ref[...]
    oi_ref[...] = img_ref[...] + pi[None]
    oa_ref[...] = aud_ref[...] + pa[None]
    pei_ref[...] = pi[None]
    pea_ref[...] = pa[None]


def kernel(image, audio, pos_image, pos_audio):
    grid = (S // BS, B)
    in_spec3 = pl.BlockSpec((1, BS, C), lambda s, b: (b, s, 0))
    in_spec2 = pl.BlockSpec((BS, C), lambda s, b: (s, 0))
    out_spec = pl.BlockSpec((1, BS, C), lambda s, b: (b, s, 0))
    out_shape = jax.ShapeDtypeStruct((B, S, C), jnp.float32)
    return pl.pallas_call(
        _pe_kernel,
        grid=grid,
        in_specs=[in_spec3, in_spec3, in_spec2, in_spec2],
        out_specs=[out_spec, out_spec, out_spec, out_spec],
        out_shape=[out_shape, out_shape, out_shape, out_shape],
    )(image, audio, pos_image, pos_audio)
